# baseline (device time: 15393 ns/iter reference)
import functools

import jax
import jax.numpy as jnp
from jax import lax
from jax.experimental import pallas as pl
from jax.experimental.pallas import tpu as pltpu

EPS = 1e-5
YDIM = 4
ZDIM = 4
N_PLANE = YDIM * ZDIM
PLANE = [(yj, zj) for yj in range(YDIM) for zj in range(ZDIM)]
N_CHUNK = 4
N_HALF = 2


def _body(
    x_hbm,
    dy_hbm,
    out_ref,
    xb,
    dyb,
    sendbuf1,
    comm1,
    sendbuf2,
    comm2,
    local_sems,
    sem1_send,
    sem1_recv,
    send_sems,
    recv_sems,
):
    my_x = lax.axis_index("x")
    my_y = lax.axis_index("y")
    my_z = lax.axis_index("z")
    k = my_y * ZDIM + my_z
    rows = xb.shape[0]
    d = xb.shape[1]
    crows = rows // N_CHUNK
    dh = d // N_HALF

    xbar = pltpu.get_barrier_semaphore()

    @functools.partial(
        pl.run_scoped, plane_bar=pltpu.SemaphoreType.REGULAR
    )
    def _(plane_bar):
        pl.semaphore_signal(
            xbar,
            inc=1,
            device_id=(1 - my_x, my_y, my_z),
            device_id_type=pl.DeviceIdType.MESH,
        )
        for j, (yj, zj) in enumerate(PLANE):

            @pl.when(j != k)
            def _(yj=yj, zj=zj):
                pl.semaphore_signal(
                    plane_bar,
                    inc=1,
                    device_id=(my_x, yj, zj),
                    device_id_type=pl.DeviceIdType.MESH,
                )

        copies = []
        for c in range(N_CHUNK):
            r0 = k * rows + c * crows
            cp_x = pltpu.make_async_copy(
                x_hbm.at[pl.ds(r0, crows), :],
                xb.at[pl.ds(c * crows, crows), :],
                local_sems.at[c, 0],
            )
            cp_dy = pltpu.make_async_copy(
                dy_hbm.at[pl.ds(r0, crows), :],
                dyb.at[pl.ds(c * crows, crows), :],
                local_sems.at[c, 1],
            )
            cp_x.start()
            cp_dy.start()
            copies.append((cp_x, cp_dy))

        dgamma = jnp.zeros((d,), jnp.float32)
        dbeta = jnp.zeros((d,), jnp.float32)
        for c in range(N_CHUNK):
            cp_x, cp_dy = copies[c]
            cp_x.wait()
            cp_dy.wait()
            x = xb[pl.ds(c * crows, crows), :]
            dy = dyb[pl.ds(c * crows, crows), :]
            mu = jnp.mean(x, axis=1, keepdims=True)
            xc = x - mu
            var = jnp.mean(xc * xc, axis=1, keepdims=True)
            xhat = xc * lax.rsqrt(var + EPS)
            dgamma = dgamma + jnp.sum(dy * xhat, axis=0)
            dbeta = dbeta + jnp.sum(dy, axis=0)
        partial = jnp.concatenate([dgamma[None, :], dbeta[None, :]], axis=0)
        sendbuf1[:, :] = partial.astype(jnp.bfloat16)

        pl.semaphore_wait(xbar, 1)
        rdma1 = []
        for h in range(N_HALF):
            r = pltpu.make_async_remote_copy(
                src_ref=sendbuf1.at[:, pl.ds(h * dh, dh)],
                dst_ref=comm1.at[:, pl.ds(h * dh, dh)],
                send_sem=sem1_send.at[h],
                recv_sem=sem1_recv.at[h],
                device_id=(1 - my_x, my_y, my_z),
                device_id_type=pl.DeviceIdType.MESH,
            )
            r.start()
            rdma1.append(r)

        first = True
        xsum_halves = []
        for h in range(N_HALF):
            rdma1[h].wait_recv()
            xsum_h = (
                sendbuf1[:, pl.ds(h * dh, dh)].astype(jnp.float32)
                + comm1[:, pl.ds(h * dh, dh)].astype(jnp.float32)
            )
            xsum_halves.append(xsum_h)
            sendbuf2[:, pl.ds(h * dh, dh)] = xsum_h.astype(jnp.bfloat16)
            if first:
                pl.semaphore_wait(plane_bar, N_PLANE - 1)
                first = False
            for j, (yj, zj) in enumerate(PLANE):

                @pl.when(j != k)
                def _(j=j, yj=yj, zj=zj, h=h):
                    r = pltpu.make_async_remote_copy(
                        src_ref=sendbuf2.at[:, pl.ds(h * dh, dh)],
                        dst_ref=comm2.at[k, :, pl.ds(h * dh, dh)],
                        send_sem=send_sems.at[h, j],
                        recv_sem=recv_sems.at[h, k],
                        device_id=(my_x, yj, zj),
                        device_id_type=pl.DeviceIdType.MESH,
                    )
                    r.start()

        xsum = jnp.concatenate(xsum_halves, axis=1)
        comm2[pl.ds(k, 1), :, :] = xsum.astype(jnp.bfloat16)[None, :, :]

        for h in range(N_HALF):
            rdma1[h].wait_send()
        for h in range(N_HALF):
            for j in range(N_PLANE):

                @pl.when(j != k)
                def _(j=j, h=h):
                    desc = pltpu.make_async_remote_copy(
                        src_ref=sendbuf2.at[:, pl.ds(h * dh, dh)],
                        dst_ref=comm2.at[j, :, pl.ds(h * dh, dh)],
                        send_sem=send_sems.at[h, j],
                        recv_sem=recv_sems.at[h, j],
                        device_id=(my_x, my_y, my_z),
                        device_id_type=pl.DeviceIdType.MESH,
                    )
                    desc.wait_send()
                    desc.wait_recv()

        out_ref[:, :] = jnp.sum(
            comm2[:, :, :].astype(jnp.float32), axis=0
        )


def kernel(x, dy, gamma):
    del gamma
    m, d = x.shape
    rows = m // N_PLANE

    return pl.pallas_call(
        _body,
        out_shape=jax.ShapeDtypeStruct((2, d), jnp.float32),
        in_specs=[
            pl.BlockSpec(memory_space=pl.ANY),
            pl.BlockSpec(memory_space=pl.ANY),
        ],
        out_specs=pl.BlockSpec(memory_space=pltpu.VMEM),
        scratch_shapes=[
            pltpu.VMEM((rows, d), jnp.float32),
            pltpu.VMEM((rows, d), jnp.float32),
            pltpu.VMEM((2, d), jnp.bfloat16),
            pltpu.VMEM((2, d), jnp.bfloat16),
            pltpu.VMEM((2, d), jnp.bfloat16),
            pltpu.VMEM((N_PLANE, 2, d), jnp.bfloat16),
            pltpu.SemaphoreType.DMA((N_CHUNK, 2)),
            pltpu.SemaphoreType.DMA((N_HALF,)),
            pltpu.SemaphoreType.DMA((N_HALF,)),
            pltpu.SemaphoreType.DMA((N_HALF, N_PLANE)),
            pltpu.SemaphoreType.DMA((N_HALF, N_PLANE)),
        ],
        compiler_params=pltpu.CompilerParams(collective_id=0),
    )(x, dy)


# device time: 14905 ns/iter; 1.0327x vs baseline; 1.0327x over previous
import functools

import jax
import jax.numpy as jnp
from jax import lax
from jax.experimental import pallas as pl
from jax.experimental.pallas import tpu as pltpu

EPS = 1e-5
YDIM = 4
ZDIM = 4
N_PLANE = YDIM * ZDIM
PLANE = [(yj, zj) for yj in range(YDIM) for zj in range(ZDIM)]


def _body(
    x_hbm,
    dy_hbm,
    out_ref,
    xb,
    dyb,
    sendbuf1,
    comm1,
    sendbuf2,
    comm2,
    local_sems,
    sem1_send,
    sem1_recv,
    send_sems,
    recv_sems,
):
    my_x = lax.axis_index("x")
    my_y = lax.axis_index("y")
    my_z = lax.axis_index("z")
    k = my_y * ZDIM + my_z
    rows = xb.shape[0]

    xbar = pltpu.get_barrier_semaphore()

    @functools.partial(
        pl.run_scoped, plane_bar=pltpu.SemaphoreType.REGULAR
    )
    def _(plane_bar):
        pl.semaphore_signal(
            xbar,
            inc=1,
            device_id=(1 - my_x, my_y, my_z),
            device_id_type=pl.DeviceIdType.MESH,
        )
        for j, (yj, zj) in enumerate(PLANE):

            @pl.when(j != k)
            def _(yj=yj, zj=zj):
                pl.semaphore_signal(
                    plane_bar,
                    inc=1,
                    device_id=(my_x, yj, zj),
                    device_id_type=pl.DeviceIdType.MESH,
                )

        cp_x = pltpu.make_async_copy(
            x_hbm.at[pl.ds(k * rows, rows), :], xb, local_sems.at[0]
        )
        cp_dy = pltpu.make_async_copy(
            dy_hbm.at[pl.ds(k * rows, rows), :], dyb, local_sems.at[1]
        )
        cp_x.start()
        cp_dy.start()
        cp_x.wait()
        cp_dy.wait()

        x = xb[:, :]
        dy = dyb[:, :]
        mu = jnp.mean(x, axis=1, keepdims=True)
        xc = x - mu
        var = jnp.mean(xc * xc, axis=1, keepdims=True)
        xhat = xc * lax.rsqrt(var + EPS)
        dgamma = jnp.sum(dy * xhat, axis=0)[None, :]
        dbeta = jnp.sum(dy, axis=0)[None, :]
        sendbuf1[:, :] = jnp.concatenate([dgamma, dbeta], axis=0).astype(
            jnp.bfloat16
        )

        pl.semaphore_wait(xbar, 1)
        rdma1 = pltpu.make_async_remote_copy(
            src_ref=sendbuf1,
            dst_ref=comm1,
            send_sem=sem1_send,
            recv_sem=sem1_recv,
            device_id=(1 - my_x, my_y, my_z),
            device_id_type=pl.DeviceIdType.MESH,
        )
        rdma1.start()
        rdma1.wait()
        xsum = sendbuf1[:, :].astype(jnp.float32) + comm1[:, :].astype(
            jnp.float32
        )
        sendbuf2[:, :] = xsum.astype(jnp.bfloat16)

        pl.semaphore_wait(plane_bar, N_PLANE - 1)
        for j, (yj, zj) in enumerate(PLANE):

            @pl.when(j != k)
            def _(j=j, yj=yj, zj=zj):
                rdma = pltpu.make_async_remote_copy(
                    src_ref=sendbuf2,
                    dst_ref=comm2.at[k],
                    send_sem=send_sems.at[j],
                    recv_sem=recv_sems.at[k],
                    device_id=(my_x, yj, zj),
                    device_id_type=pl.DeviceIdType.MESH,
                )
                rdma.start()

        comm2[pl.ds(k, 1), :, :] = xsum.astype(jnp.bfloat16)[None, :, :]

        for j in range(N_PLANE):

            @pl.when(j != k)
            def _(j=j):
                desc = pltpu.make_async_remote_copy(
                    src_ref=sendbuf2,
                    dst_ref=comm2.at[j],
                    send_sem=send_sems.at[j],
                    recv_sem=recv_sems.at[j],
                    device_id=(my_x, my_y, my_z),
                    device_id_type=pl.DeviceIdType.MESH,
                )
                desc.wait_send()
                desc.wait_recv()

        out_ref[:, :] = jnp.sum(
            comm2[:, :, :].astype(jnp.float32), axis=0
        )


def kernel(x, dy, gamma):
    del gamma
    m, d = x.shape
    rows = m // N_PLANE

    return pl.pallas_call(
        _body,
        out_shape=jax.ShapeDtypeStruct((2, d), jnp.float32),
        in_specs=[
            pl.BlockSpec(memory_space=pl.ANY),
            pl.BlockSpec(memory_space=pl.ANY),
        ],
        out_specs=pl.BlockSpec(memory_space=pltpu.VMEM),
        scratch_shapes=[
            pltpu.VMEM((rows, d), jnp.float32),
            pltpu.VMEM((rows, d), jnp.float32),
            pltpu.VMEM((2, d), jnp.bfloat16),
            pltpu.VMEM((2, d), jnp.bfloat16),
            pltpu.VMEM((2, d), jnp.bfloat16),
            pltpu.VMEM((N_PLANE, 2, d), jnp.bfloat16),
            pltpu.SemaphoreType.DMA((2,)),
            pltpu.SemaphoreType.DMA,
            pltpu.SemaphoreType.DMA,
            pltpu.SemaphoreType.DMA((N_PLANE,)),
            pltpu.SemaphoreType.DMA((N_PLANE,)),
        ],
        compiler_params=pltpu.CompilerParams(collective_id=0),
    )(x, dy)


# device time: 14688 ns/iter; 1.0480x vs baseline; 1.0148x over previous
import functools

import jax
import jax.numpy as jnp
from jax import lax
from jax.experimental import pallas as pl
from jax.experimental.pallas import tpu as pltpu

EPS = 1e-5
YDIM = 4
ZDIM = 4
N_PLANE = YDIM * ZDIM
PLANE = [(yj, zj) for yj in range(YDIM) for zj in range(ZDIM)]


def _body(
    x_hbm,
    dy_hbm,
    out_ref,
    xb,
    dyb,
    sendbuf1,
    comm1,
    sendbuf2,
    comm2,
    local_sems,
    sem1_send,
    sem1_recv,
    send_sems,
    recv_sems,
):
    my_x = lax.axis_index("x")
    my_y = lax.axis_index("y")
    my_z = lax.axis_index("z")
    k = my_y * ZDIM + my_z
    rows = xb.shape[0]

    xbar = pltpu.get_barrier_semaphore()

    @functools.partial(
        pl.run_scoped, plane_bar=pltpu.SemaphoreType.REGULAR
    )
    def _(plane_bar):
        pl.semaphore_signal(
            xbar,
            inc=1,
            device_id=(1 - my_x, my_y, my_z),
            device_id_type=pl.DeviceIdType.MESH,
        )
        for j, (yj, zj) in enumerate(PLANE):

            @pl.when(j != k)
            def _(yj=yj, zj=zj):
                pl.semaphore_signal(
                    plane_bar,
                    inc=1,
                    device_id=(my_x, yj, zj),
                    device_id_type=pl.DeviceIdType.MESH,
                )

        half = rows // 2
        copies = []
        for c in range(2):
            r0 = k * rows + c * half
            cp_x = pltpu.make_async_copy(
                x_hbm.at[pl.ds(r0, half), :],
                xb.at[pl.ds(c * half, half), :],
                local_sems.at[c, 0],
            )
            cp_dy = pltpu.make_async_copy(
                dy_hbm.at[pl.ds(r0, half), :],
                dyb.at[pl.ds(c * half, half), :],
                local_sems.at[c, 1],
            )
            cp_x.start()
            cp_dy.start()
            copies.append((cp_x, cp_dy))

        dgamma = jnp.zeros((1, xb.shape[1]), jnp.float32)
        dbeta = jnp.zeros((1, xb.shape[1]), jnp.float32)
        for c in range(2):
            cp_x, cp_dy = copies[c]
            cp_x.wait()
            cp_dy.wait()
            x = xb[pl.ds(c * half, half), :]
            dy = dyb[pl.ds(c * half, half), :]
            mu = jnp.mean(x, axis=1, keepdims=True)
            xc = x - mu
            var = jnp.mean(xc * xc, axis=1, keepdims=True)
            xhat = xc * lax.rsqrt(var + EPS)
            dgamma = dgamma + jnp.sum(dy * xhat, axis=0)[None, :]
            dbeta = dbeta + jnp.sum(dy, axis=0)[None, :]
        sendbuf1[:, :] = jnp.concatenate([dgamma, dbeta], axis=0).astype(
            jnp.bfloat16
        )

        pl.semaphore_wait(xbar, 1)
        rdma1 = pltpu.make_async_remote_copy(
            src_ref=sendbuf1,
            dst_ref=comm1,
            send_sem=sem1_send,
            recv_sem=sem1_recv,
            device_id=(1 - my_x, my_y, my_z),
            device_id_type=pl.DeviceIdType.MESH,
        )
        rdma1.start()
        rdma1.wait()
        xsum = sendbuf1[:, :].astype(jnp.float32) + comm1[:, :].astype(
            jnp.float32
        )
        sendbuf2[:, :] = xsum.astype(jnp.bfloat16)

        pl.semaphore_wait(plane_bar, N_PLANE - 1)
        for j, (yj, zj) in enumerate(PLANE):

            @pl.when(j != k)
            def _(j=j, yj=yj, zj=zj):
                rdma = pltpu.make_async_remote_copy(
                    src_ref=sendbuf2,
                    dst_ref=comm2.at[k],
                    send_sem=send_sems.at[j],
                    recv_sem=recv_sems.at[k],
                    device_id=(my_x, yj, zj),
                    device_id_type=pl.DeviceIdType.MESH,
                )
                rdma.start()

        comm2[pl.ds(k, 1), :, :] = xsum.astype(jnp.bfloat16)[None, :, :]

        for j in range(N_PLANE):

            @pl.when(j != k)
            def _(j=j):
                desc = pltpu.make_async_remote_copy(
                    src_ref=sendbuf2,
                    dst_ref=comm2.at[j],
                    send_sem=send_sems.at[j],
                    recv_sem=recv_sems.at[j],
                    device_id=(my_x, my_y, my_z),
                    device_id_type=pl.DeviceIdType.MESH,
                )
                desc.wait_send()
                desc.wait_recv()

        out_ref[:, :] = jnp.sum(
            comm2[:, :, :].astype(jnp.float32), axis=0
        )


def kernel(x, dy, gamma):
    del gamma
    m, d = x.shape
    rows = m // N_PLANE

    return pl.pallas_call(
        _body,
        out_shape=jax.ShapeDtypeStruct((2, d), jnp.float32),
        in_specs=[
            pl.BlockSpec(memory_space=pl.ANY),
            pl.BlockSpec(memory_space=pl.ANY),
        ],
        out_specs=pl.BlockSpec(memory_space=pltpu.VMEM),
        scratch_shapes=[
            pltpu.VMEM((rows, d), jnp.float32),
            pltpu.VMEM((rows, d), jnp.float32),
            pltpu.VMEM((2, d), jnp.bfloat16),
            pltpu.VMEM((2, d), jnp.bfloat16),
            pltpu.VMEM((2, d), jnp.bfloat16),
            pltpu.VMEM((N_PLANE, 2, d), jnp.bfloat16),
            pltpu.SemaphoreType.DMA((2, 2)),
            pltpu.SemaphoreType.DMA,
            pltpu.SemaphoreType.DMA,
            pltpu.SemaphoreType.DMA((N_PLANE,)),
            pltpu.SemaphoreType.DMA((N_PLANE,)),
        ],
        compiler_params=pltpu.CompilerParams(collective_id=0),
    )(x, dy)


# device time: 14556 ns/iter; 1.0575x vs baseline; 1.0091x over previous
import functools

import jax
import jax.numpy as jnp
from jax import lax
from jax.experimental import pallas as pl
from jax.experimental.pallas import tpu as pltpu

EPS = 1e-5
YDIM = 4
ZDIM = 4
N_PLANE = YDIM * ZDIM
PLANE = [(yj, zj) for yj in range(YDIM) for zj in range(ZDIM)]


def _body(
    x_hbm,
    dy_hbm,
    out_ref,
    xb,
    dyb,
    sendbuf1,
    comm1,
    sendbuf2,
    comm2,
    local_sems,
    sem1_send,
    sem1_recv,
    send_sems,
    recv_sems,
):
    my_x = lax.axis_index("x")
    my_y = lax.axis_index("y")
    my_z = lax.axis_index("z")
    k = my_y * ZDIM + my_z
    rows = xb.shape[0]

    xbar = pltpu.get_barrier_semaphore()

    @functools.partial(
        pl.run_scoped, plane_bar=pltpu.SemaphoreType.REGULAR
    )
    def _(plane_bar):
        pl.semaphore_signal(
            xbar,
            inc=1,
            device_id=(1 - my_x, my_y, my_z),
            device_id_type=pl.DeviceIdType.MESH,
        )
        for j, (yj, zj) in enumerate(PLANE):

            @pl.when(j != k)
            def _(yj=yj, zj=zj):
                pl.semaphore_signal(
                    plane_bar,
                    inc=1,
                    device_id=(my_x, yj, zj),
                    device_id_type=pl.DeviceIdType.MESH,
                )

        half = rows // 2
        copies = []
        for c in range(2):
            r0 = k * rows + c * half
            cp_x = pltpu.make_async_copy(
                x_hbm.at[pl.ds(r0, half), :],
                xb.at[pl.ds(c * half, half), :],
                local_sems.at[c, 0],
            )
            cp_dy = pltpu.make_async_copy(
                dy_hbm.at[pl.ds(r0, half), :],
                dyb.at[pl.ds(c * half, half), :],
                local_sems.at[c, 1],
            )
            cp_x.start()
            cp_dy.start()
            copies.append((cp_x, cp_dy))

        dgamma = jnp.zeros((1, xb.shape[1]), jnp.float32)
        dbeta = jnp.zeros((1, xb.shape[1]), jnp.float32)
        for c in range(2):
            cp_x, cp_dy = copies[c]
            cp_x.wait()
            cp_dy.wait()
            x = xb[pl.ds(c * half, half), :]
            dy = dyb[pl.ds(c * half, half), :]
            mu = jnp.mean(x, axis=1, keepdims=True)
            xc = x - mu
            var = jnp.mean(xc * xc, axis=1, keepdims=True)
            xhat = xc * lax.rsqrt(var + EPS)
            dgamma = dgamma + jnp.sum(dy * xhat, axis=0)[None, :]
            dbeta = dbeta + jnp.sum(dy, axis=0)[None, :]
        sendbuf1[:, :] = jnp.concatenate([dgamma, dbeta], axis=0).astype(
            jnp.bfloat16
        )

        pl.semaphore_wait(xbar, 1)
        rdma1 = pltpu.make_async_remote_copy(
            src_ref=sendbuf1,
            dst_ref=comm1,
            send_sem=sem1_send,
            recv_sem=sem1_recv,
            device_id=(1 - my_x, my_y, my_z),
            device_id_type=pl.DeviceIdType.MESH,
        )
        rdma1.start()
        rdma1.wait()
        xsum = sendbuf1[:, :].astype(jnp.float32) + comm1[:, :].astype(
            jnp.float32
        )
        sendbuf2[:, :] = xsum.astype(jnp.bfloat16)

        pl.semaphore_wait(plane_bar, N_PLANE - 1)
        for j, (yj, zj) in enumerate(PLANE):

            @pl.when(j != k)
            def _(j=j, yj=yj, zj=zj):
                rdma = pltpu.make_async_remote_copy(
                    src_ref=sendbuf2,
                    dst_ref=comm2.at[k],
                    send_sem=send_sems.at[j],
                    recv_sem=recv_sems.at[k],
                    device_id=(my_x, yj, zj),
                    device_id_type=pl.DeviceIdType.MESH,
                )
                rdma.start()

        out_ref[:, :] = xsum

        for j in range(N_PLANE):

            @pl.when(j != k)
            def _(j=j):
                desc = pltpu.make_async_remote_copy(
                    src_ref=sendbuf2,
                    dst_ref=comm2.at[j],
                    send_sem=send_sems.at[j],
                    recv_sem=recv_sems.at[j],
                    device_id=(my_x, my_y, my_z),
                    device_id_type=pl.DeviceIdType.MESH,
                )
                desc.wait_send()
                desc.wait_recv()
                out_ref[:, :] = out_ref[:, :] + comm2[j, :, :].astype(
                    jnp.float32
                )


def kernel(x, dy, gamma):
    del gamma
    m, d = x.shape
    rows = m // N_PLANE

    return pl.pallas_call(
        _body,
        out_shape=jax.ShapeDtypeStruct((2, d), jnp.float32),
        in_specs=[
            pl.BlockSpec(memory_space=pl.ANY),
            pl.BlockSpec(memory_space=pl.ANY),
        ],
        out_specs=pl.BlockSpec(memory_space=pltpu.VMEM),
        scratch_shapes=[
            pltpu.VMEM((rows, d), jnp.float32),
            pltpu.VMEM((rows, d), jnp.float32),
            pltpu.VMEM((2, d), jnp.bfloat16),
            pltpu.VMEM((2, d), jnp.bfloat16),
            pltpu.VMEM((2, d), jnp.bfloat16),
            pltpu.VMEM((N_PLANE, 2, d), jnp.bfloat16),
            pltpu.SemaphoreType.DMA((2, 2)),
            pltpu.SemaphoreType.DMA,
            pltpu.SemaphoreType.DMA,
            pltpu.SemaphoreType.DMA((N_PLANE,)),
            pltpu.SemaphoreType.DMA((N_PLANE,)),
        ],
        compiler_params=pltpu.CompilerParams(collective_id=0),
    )(x, dy)
